# baseline (speedup 1.0000x reference)
import jax
import jax.numpy as jnp
from jax.experimental import pallas as pl
from jax.experimental.pallas import tpu as pltpu

_BLK = 256


def _elu(x):
    return jnp.where(x > 0, x, jnp.exp(jnp.minimum(x, 0.0)) - 1.0)


def _stage_feats(f, asn, feats_s, featsb_s, esen_s, esent_s):
    feats_s[...] = f
    featsb_s[...] = f.astype(jnp.bfloat16)
    esen = jnp.dot(f, asn, preferred_element_type=jnp.float32)
    esen_s[...] = esen
    esent_s[...] = jnp.transpose(esen)


def _body(scale_ref, x_ref, idx_ref, a_ref, down_ref, orig_ref,
          wup_ref, asn_up_ref, wend_ref, asn_end_ref, out_ref,
          feats_s, featsb_s, esen_s, esent_s, x1_s):
    No, F = x_ref.shape[1], x_ref.shape[2]
    Nn = a_ref.shape[2]
    H = asn_up_ref.shape[1] // 2
    C = wup_ref.shape[1] // H
    ph = pl.program_id(1)
    i = pl.program_id(2)
    row0 = i * _BLK

    @pl.when(jnp.logical_and(ph == 0, i == 0))
    def _():
        x = x_ref[0]
        idx = idx_ref[0, 0, :]
        rows = jax.lax.broadcasted_iota(jnp.int32, (Nn, No), 0)
        onehot = (rows == idx[None, :]).astype(jnp.float32)
        xu = jnp.dot(onehot, x, preferred_element_type=jnp.float32)
        f = jnp.dot(xu, wup_ref[...], preferred_element_type=jnp.float32)
        _stage_feats(f, asn_up_ref[...], feats_s, featsb_s, esen_s, esent_s)

    @pl.when(jnp.logical_and(ph == 1, i == 0))
    def _():
        f = (jnp.dot(x1_s[...], wend_ref[:F, :],
                     preferred_element_type=jnp.float32)
             + jnp.dot(orig_ref[0], wend_ref[F:, :],
                       preferred_element_type=jnp.float32))
        _stage_feats(f, asn_end_ref[...], feats_s, featsb_s, esen_s, esent_s)

    a_blk = a_ref[0]
    ri = jax.lax.broadcasted_iota(jnp.int32, (_BLK, Nn), 0) + row0
    ci = jax.lax.broadcasted_iota(jnp.int32, (_BLK, Nn), 1)
    edge = jnp.logical_or(a_blk > 0.5, ri == ci)
    neg_mask = jnp.where(edge, 0.0, -1e9).astype(jnp.float32)

    inv_h = 1.0 / H
    acc = jnp.zeros((_BLK, C), jnp.float32)
    for h in range(H):
        es = esen_s[pl.ds(row0, _BLK), h:h + 1]
        en = esent_s[H + h:H + h + 1, :]
        t = es + en
        p = jnp.exp2(jnp.maximum(t, 0.2 * t) + neg_mask)
        pb = p.astype(jnp.bfloat16)
        s = jnp.sum(p, axis=1, keepdims=True)
        acc = acc + jnp.dot(pb, featsb_s[:, h * C:(h + 1) * C],
                            preferred_element_type=jnp.float32) * (inv_h / s)

    @pl.when(ph == 0)
    def _():
        r = _elu(acc) + down_ref[0]
        x1_s[pl.ds(row0, _BLK), :] = r
        out_ref[0] = r

    @pl.when(ph == 1)
    def _():
        out_ref[0] = _elu(acc) * scale_ref[0]


def kernel(X, orig_X, l_n, idx0, A0, down0, action, W_up, a_s_up, a_n_up,
           W_end, a_s_end, a_n_end):
    B, No, F = X.shape
    Nn = A0.shape[1]
    H, C = a_s_up.shape
    nblk = Nn // _BLK
    idx3 = idx0.astype(jnp.int32).reshape(B, 1, No)
    wup = W_up.reshape(F, H * C)
    wend = W_end.reshape(2 * F, H * C)

    log2e = 1.4426950408889634
    eye = jnp.eye(H, dtype=jnp.float32)
    asn_up = jnp.concatenate(
        [(a_s_up[:, :, None] * eye[:, None, :]).reshape(H * C, H),
         (a_n_up[:, :, None] * eye[:, None, :]).reshape(H * C, H)],
        axis=1) * log2e
    asn_end = jnp.concatenate(
        [(a_s_end[:, :, None] * eye[:, None, :]).reshape(H * C, H),
         (a_n_end[:, :, None] * eye[:, None, :]).reshape(H * C, H)],
        axis=1) * log2e

    scale = (jnp.asarray(l_n) / 1).astype(jnp.float32).reshape(1)

    full = lambda *shape: pl.BlockSpec(shape, lambda b, p, i: (0,) * len(shape))
    out = pl.pallas_call(
        _body,
        grid=(B, 2, nblk),
        in_specs=[
            pl.BlockSpec(memory_space=pltpu.SMEM),
            pl.BlockSpec((1, No, F), lambda b, p, i: (b, 0, 0)),
            pl.BlockSpec((1, 1, No), lambda b, p, i: (b, 0, 0)),
            pl.BlockSpec((1, _BLK, Nn), lambda b, p, i: (b, i, 0)),
            pl.BlockSpec((1, _BLK, F), lambda b, p, i: (b, i, 0)),
            pl.BlockSpec((1, Nn, F), lambda b, p, i: (b, 0, 0)),
            full(F, H * C),
            full(H * C, 2 * H),
            full(2 * F, H * C),
            full(H * C, 2 * H),
        ],
        out_specs=pl.BlockSpec((1, _BLK, F), lambda b, p, i: (b, i, 0)),
        out_shape=jax.ShapeDtypeStruct((B, Nn, F), jnp.float32),
        scratch_shapes=[
            pltpu.VMEM((Nn, H * C), jnp.float32),
            pltpu.VMEM((Nn, H * C), jnp.bfloat16),
            pltpu.VMEM((Nn, 2 * H), jnp.float32),
            pltpu.VMEM((2 * H, Nn), jnp.float32),
            pltpu.VMEM((Nn, F), jnp.float32),
        ],
    )(scale, X, idx3, A0, down0, orig_X, wup, asn_up, wend, asn_end)
    return out
